# baseline (speedup 1.0000x reference)
import functools

import jax
import jax.numpy as jnp
from jax import lax
from jax.experimental import pallas as pl
from jax.experimental.pallas import tpu as pltpu
from jax.experimental.pallas import tpu_sc as plsc

N = 8192
H = 128
GRID = 4
G = GRID * GRID
AREA_SPAN = 2.0
HALF = AREA_SPAN / 2.0
EPS = 0.01
THR = HALF - EPS
INV_CELL = GRID / AREA_SPAN

NW = 32
IPW = N // NW
SB = 16
NSB = IPW // SB
JC = 256
LANES = 16
HC = H // LANES
ACC = SB * G * H


def _sc_body(ht_hbm, px_hbm, py_hbm, rs_hbm, re_hbm, out_hbm,
             htc, pxc, pyc, rsw, rew, pxw, pyw, acc0, acc1, sem,
             semw0, semw1):
    cid = lax.axis_index("c")
    sid = lax.axis_index("s")
    wid = sid * 2 + cid
    i_base = wid * IPW
    iota = lax.iota(jnp.int32, LANES)
    ioff = iota * (G * H)
    doff = ACC + iota * H
    zeros16 = jnp.zeros((LANES,), jnp.float32)
    accs = (acc0, acc1)
    semws = (semw0, semw1)

    m1 = pltpu.async_copy(rs_hbm.at[pl.ds(i_base, IPW)], rsw, sem)
    m2 = pltpu.async_copy(re_hbm.at[pl.ds(i_base, IPW)], rew, sem)
    m3 = pltpu.async_copy(px_hbm.at[pl.ds(i_base, IPW)], pxw, sem)
    m4 = pltpu.async_copy(py_hbm.at[pl.ds(i_base, IPW)], pyw, sem)
    m1.wait()
    m2.wait()
    m3.wait()
    m4.wait()

    def run_subblock(b, acc, semw, drain_prev):
        i0 = i_base + b * SB
        rsv = rsw[pl.ds(b * SB, LANES)]
        rev = rew[pl.ds(b * SB, LANES)]
        xiv = pxw[pl.ds(b * SB, LANES)]
        yiv = pyw[pl.ds(b * SB, LANES)]
        iiv = i0 + iota
        rs0 = rsv[0]
        re_last = rev[LANES - 1]
        jb0 = (rs0 // 8) * 8
        nch = (re_last - jb0 + JC - 1) // JC

        c1 = pltpu.async_copy(ht_hbm.at[pl.ds(jb0 * H, JC * H)], htc, sem)
        c2 = pltpu.async_copy(px_hbm.at[pl.ds(jb0, JC)],
                              pxc.at[pl.ds(0, JC)], sem)
        c3 = pltpu.async_copy(py_hbm.at[pl.ds(jb0, JC)],
                              pyc.at[pl.ds(0, JC)], sem)
        drain_prev()

        def zbody(k, _):
            for u in range(16):
                acc[pl.ds((k * 16 + u) * LANES, LANES)] = zeros16
            return 0

        lax.fori_loop(0, ACC // LANES // 16, zbody, 0)
        c1.wait()
        c2.wait()
        c3.wait()

        def compute_chunk(jb):
            lo = jnp.clip(rs0 - jb, 0, JC)
            hi = jnp.clip(re_last - jb, lo, JC)

            def do_j(jl):
                xj = pxc[pl.ds(jl, LANES)][0]
                yj = pyc[pl.ds(jl, LANES)][0]
                jg = jb + jl
                relx = xj - xiv
                rely = yj - yiv
                okv = ((relx < THR) & (relx > -THR)
                       & (rely < THR) & (rely > -THR)
                       & (jg != iiv) & (jg >= rsv) & (jg < rev))
                gxv = ((relx + HALF) * INV_CELL).astype(jnp.int32)
                gyv = ((rely + HALF) * INV_CELL).astype(jnp.int32)
                offv = jnp.where(okv, ioff + (gxv * GRID + gyv) * H, doff)
                hb = jl * H
                vs = [htc[pl.ds(hb + c * LANES, LANES)] for c in range(HC)]
                for l in range(LANES):
                    ol = offv[l]
                    for c in range(HC):
                        plsc.addupdate(
                            acc.at[pl.ds(ol + c * LANES, LANES)], vs[c])

            n2 = (hi - lo) // 2

            def per_j2(k, _):
                jl = lo + k * 2
                do_j(jl)
                do_j(jl + 1)
                return 0

            lax.fori_loop(0, n2, per_j2, 0)

            @pl.when(lo + n2 * 2 < hi)
            def _():
                do_j(hi - 1)

        compute_chunk(jb0)

        def chunk(ci, _):
            jb = jb0 + ci * JC
            d1 = pltpu.async_copy(ht_hbm.at[pl.ds(jb * H, JC * H)], htc, sem)
            d2 = pltpu.async_copy(px_hbm.at[pl.ds(jb, JC)],
                                  pxc.at[pl.ds(0, JC)], sem)
            d3 = pltpu.async_copy(py_hbm.at[pl.ds(jb, JC)],
                                  pyc.at[pl.ds(0, JC)], sem)
            d1.wait()
            d2.wait()
            d3.wait()
            compute_chunk(jb)
            return 0

        lax.fori_loop(1, nch, chunk, 0)
        pltpu.async_copy(acc.at[pl.ds(0, ACC)],
                         out_hbm.at[pl.ds(i0 * G * H, ACC)], semw)

    def wb_drain(b, acc, semw):
        i0p = i_base + b * SB
        pltpu.make_async_copy(acc.at[pl.ds(0, ACC)],
                              out_hbm.at[pl.ds(i0p * G * H, ACC)],
                              semw).wait()

    def subpair(bp, _):
        for par in range(2):
            b = bp * 2 + par
            acc = accs[par]
            semw = semws[par]

            def drain_prev(b=b, acc=acc, semw=semw):
                @pl.when(b >= 2)
                def _():
                    wb_drain(b - 2, acc, semw)

            run_subblock(b, acc, semw, drain_prev)
        return 0

    lax.fori_loop(0, NSB // 2, subpair, 0)
    wb_drain(NSB - 2, accs[0], semws[0])
    wb_drain(NSB - 1, accs[1], semws[1])


_sc_pool = functools.partial(
    pl.kernel,
    out_type=jax.ShapeDtypeStruct((N * G * H,), jnp.float32),
    mesh=plsc.VectorSubcoreMesh(core_axis_name="c", subcore_axis_name="s"),
    scratch_types=[
        pltpu.VMEM((JC * H,), jnp.float32),
        pltpu.VMEM((JC + LANES,), jnp.float32),
        pltpu.VMEM((JC + LANES,), jnp.float32),
        pltpu.VMEM((IPW,), jnp.int32),
        pltpu.VMEM((IPW,), jnp.int32),
        pltpu.VMEM((IPW,), jnp.float32),
        pltpu.VMEM((IPW,), jnp.float32),
        pltpu.VMEM((ACC + SB * H,), jnp.float32),
        pltpu.VMEM((ACC + SB * H,), jnp.float32),
        pltpu.SemaphoreType.DMA,
        pltpu.SemaphoreType.DMA,
        pltpu.SemaphoreType.DMA,
    ],
)(_sc_body)


def kernel(ht, pos_t, same_scene_mask):
    ht2 = ht.reshape(N, H)
    pos = pos_t.reshape(N, 2)
    scene = same_scene_mask.reshape(N)
    idx = jnp.arange(N, dtype=jnp.int32)
    prev_ne = jnp.concatenate(
        [jnp.ones((1,), bool), scene[1:] != scene[:-1]])
    next_ne = jnp.concatenate(
        [scene[1:] != scene[:-1], jnp.ones((1,), bool)])
    rs = lax.associative_scan(jnp.maximum, jnp.where(prev_ne, idx, 0))
    re_ = lax.associative_scan(jnp.minimum, jnp.where(next_ne, idx + 1, N),
                               reverse=True)
    zf = jnp.zeros((JC,), jnp.float32)
    ht_pad = jnp.concatenate(
        [ht2, jnp.zeros((JC, H), ht2.dtype)], axis=0).reshape((N + JC) * H)
    px_pad = jnp.concatenate([pos[:, 0], zf])
    py_pad = jnp.concatenate([pos[:, 1], zf])
    out = _sc_pool(ht_pad, px_pad, py_pad, rs, re_)
    return out.reshape(N, G, H)


# speedup vs baseline: 1.0076x; 1.0076x over previous
import functools

import jax
import jax.numpy as jnp
from jax import lax
from jax.experimental import pallas as pl
from jax.experimental.pallas import tpu as pltpu
from jax.experimental.pallas import tpu_sc as plsc

N = 8192
H = 128
GRID = 4
G = GRID * GRID
AREA_SPAN = 2.0
HALF = AREA_SPAN / 2.0
EPS = 0.01
THR = HALF - EPS
INV_CELL = GRID / AREA_SPAN

NW = 32
IPW = N // NW
SB = 16
NSB = IPW // SB
JC = 256
LANES = 16
HC = H // LANES
ACC = SB * G * H


def _sc_body(ht_hbm, px_hbm, py_hbm, rs_hbm, re_hbm, out_hbm,
             htc, pxc, pyc, rsw, rew, pxw, pyw, acc0, acc1, sem,
             semw0, semw1):
    cid = lax.axis_index("c")
    sid = lax.axis_index("s")
    wid = sid * 2 + cid
    i_base = wid * IPW
    iota = lax.iota(jnp.int32, LANES)
    ioff = iota * (G * H)
    doff = ACC + iota * H
    zeros16 = jnp.zeros((LANES,), jnp.float32)
    accs = (acc0, acc1)
    semws = (semw0, semw1)

    m1 = pltpu.async_copy(rs_hbm.at[pl.ds(i_base, IPW)], rsw, sem)
    m2 = pltpu.async_copy(re_hbm.at[pl.ds(i_base, IPW)], rew, sem)
    m3 = pltpu.async_copy(px_hbm.at[pl.ds(i_base, IPW)], pxw, sem)
    m4 = pltpu.async_copy(py_hbm.at[pl.ds(i_base, IPW)], pyw, sem)
    m1.wait()
    m2.wait()
    m3.wait()
    m4.wait()

    def run_subblock(b, acc, semw, drain_prev):
        i0 = i_base + b * SB
        rsv = rsw[pl.ds(b * SB, LANES)]
        rev = rew[pl.ds(b * SB, LANES)]
        xiv = pxw[pl.ds(b * SB, LANES)]
        yiv = pyw[pl.ds(b * SB, LANES)]
        iiv = i0 + iota
        rs0 = rsv[0]
        re_last = rev[LANES - 1]
        jb0 = (rs0 // 8) * 8
        nch = (re_last - jb0 + JC - 1) // JC

        c1 = pltpu.async_copy(ht_hbm.at[pl.ds(jb0 * H, JC * H)], htc, sem)
        c2 = pltpu.async_copy(px_hbm.at[pl.ds(jb0, JC)],
                              pxc.at[pl.ds(0, JC)], sem)
        c3 = pltpu.async_copy(py_hbm.at[pl.ds(jb0, JC)],
                              pyc.at[pl.ds(0, JC)], sem)
        drain_prev()

        def zbody(k, _):
            for u in range(16):
                acc[pl.ds((k * 16 + u) * LANES, LANES)] = zeros16
            return 0

        lax.fori_loop(0, ACC // LANES // 16, zbody, 0)
        c1.wait()
        c2.wait()
        c3.wait()

        def compute_chunk(jb):
            lo = jnp.clip(rs0 - jb, 0, JC)
            hi = jnp.clip(re_last - jb, lo, JC)

            def per_j(jl, _):
                xj = pxc[pl.ds(jl, LANES)][0]
                yj = pyc[pl.ds(jl, LANES)][0]
                jg = jb + jl
                relx = xj - xiv
                rely = yj - yiv
                okv = ((relx < THR) & (relx > -THR)
                       & (rely < THR) & (rely > -THR)
                       & (jg != iiv) & (jg >= rsv) & (jg < rev))
                gxv = ((relx + HALF) * INV_CELL).astype(jnp.int32)
                gyv = ((rely + HALF) * INV_CELL).astype(jnp.int32)
                offv = jnp.where(okv, ioff + (gxv * GRID + gyv) * H, doff)
                hb = jl * H
                vs = [htc[pl.ds(hb + c * LANES, LANES)] for c in range(HC)]
                for l in range(LANES):
                    ol = offv[l]
                    for c in range(HC):
                        plsc.addupdate(
                            acc.at[pl.ds(ol + c * LANES, LANES)], vs[c])
                return 0

            lax.fori_loop(lo, hi, per_j, 0)

        compute_chunk(jb0)

        def chunk(ci, _):
            jb = jb0 + ci * JC
            d1 = pltpu.async_copy(ht_hbm.at[pl.ds(jb * H, JC * H)], htc, sem)
            d2 = pltpu.async_copy(px_hbm.at[pl.ds(jb, JC)],
                                  pxc.at[pl.ds(0, JC)], sem)
            d3 = pltpu.async_copy(py_hbm.at[pl.ds(jb, JC)],
                                  pyc.at[pl.ds(0, JC)], sem)
            d1.wait()
            d2.wait()
            d3.wait()
            compute_chunk(jb)
            return 0

        lax.fori_loop(1, nch, chunk, 0)
        pltpu.async_copy(acc.at[pl.ds(0, ACC)],
                         out_hbm.at[pl.ds(i0 * G * H, ACC)], semw)

    def wb_drain(b, acc, semw):
        i0p = i_base + b * SB
        pltpu.make_async_copy(acc.at[pl.ds(0, ACC)],
                              out_hbm.at[pl.ds(i0p * G * H, ACC)],
                              semw).wait()

    def subpair(bp, _):
        for par in range(2):
            b = bp * 2 + par
            acc = accs[par]
            semw = semws[par]

            def drain_prev(b=b, acc=acc, semw=semw):
                @pl.when(b >= 2)
                def _():
                    wb_drain(b - 2, acc, semw)

            run_subblock(b, acc, semw, drain_prev)
        return 0

    lax.fori_loop(0, NSB // 2, subpair, 0)
    wb_drain(NSB - 2, accs[0], semws[0])
    wb_drain(NSB - 1, accs[1], semws[1])


_sc_pool = functools.partial(
    pl.kernel,
    out_type=jax.ShapeDtypeStruct((N * G * H,), jnp.float32),
    mesh=plsc.VectorSubcoreMesh(core_axis_name="c", subcore_axis_name="s"),
    scratch_types=[
        pltpu.VMEM((JC * H,), jnp.float32),
        pltpu.VMEM((JC + LANES,), jnp.float32),
        pltpu.VMEM((JC + LANES,), jnp.float32),
        pltpu.VMEM((IPW,), jnp.int32),
        pltpu.VMEM((IPW,), jnp.int32),
        pltpu.VMEM((IPW,), jnp.float32),
        pltpu.VMEM((IPW,), jnp.float32),
        pltpu.VMEM((ACC + SB * H,), jnp.float32),
        pltpu.VMEM((ACC + SB * H,), jnp.float32),
        pltpu.SemaphoreType.DMA,
        pltpu.SemaphoreType.DMA,
        pltpu.SemaphoreType.DMA,
    ],
)(_sc_body)


def kernel(ht, pos_t, same_scene_mask):
    ht2 = ht.reshape(N, H)
    pos = pos_t.reshape(N, 2)
    scene = same_scene_mask.reshape(N)
    idx = jnp.arange(N, dtype=jnp.int32)
    prev_ne = jnp.concatenate(
        [jnp.ones((1,), bool), scene[1:] != scene[:-1]])
    next_ne = jnp.concatenate(
        [scene[1:] != scene[:-1], jnp.ones((1,), bool)])
    rs = lax.associative_scan(jnp.maximum, jnp.where(prev_ne, idx, 0))
    re_ = lax.associative_scan(jnp.minimum, jnp.where(next_ne, idx + 1, N),
                               reverse=True)
    zf = jnp.zeros((JC,), jnp.float32)
    ht_pad = jnp.concatenate(
        [ht2, jnp.zeros((JC, H), ht2.dtype)], axis=0).reshape((N + JC) * H)
    px_pad = jnp.concatenate([pos[:, 0], zf])
    py_pad = jnp.concatenate([pos[:, 1], zf])
    out = _sc_pool(ht_pad, px_pad, py_pad, rs, re_)
    return out.reshape(N, G, H)
